# baseline (speedup 1.0000x reference)
import functools

import jax
import jax.numpy as jnp
from jax import lax
from jax.experimental import pallas as pl
from jax.experimental.pallas import tpu as pltpu
from jax.experimental.pallas import tpu_sc as plsc

B = 16384
D = 16
NC = 2
NS = 16
V = 1000000
SLAB = 2048
NFULL = 488
TAIL_C0 = NFULL * SLAB
TAIL_W = 640
CAP = 96
NBKT = 31
RCH = 16
SHIFT = 11

_mesh = plsc.VectorSubcoreMesh(core_axis_name="c", subcore_axis_name="s")


@functools.partial(
    pl.kernel,
    out_type=jax.ShapeDtypeStruct((2 * B,), jnp.float32),
    mesh=_mesh,
    scratch_types=[
        pltpu.VMEM((B,), jnp.int32),
        pltpu.VMEM((D, SLAB), jnp.float32),
        pltpu.VMEM((D, SLAB), jnp.float32),
        pltpu.VMEM((NBKT * CAP,), jnp.int32),
        pltpu.VMEM((NBKT * CAP,), jnp.int32),
        pltpu.VMEM((32,), jnp.int32),
        pltpu.VMEM((RCH, 128), jnp.float32),
        pltpu.VMEM((RCH, 128), jnp.int32),
        pltpu.VMEM((2 * D + 1, D), jnp.float32),
        pltpu.VMEM((1024,), jnp.float32),
        pltpu.VMEM_SHARED((B,), jnp.float32),
        pltpu.SemaphoreType.DMA,
        pltpu.SemaphoreType.DMA,
    ],
    compiler_params=pltpu.CompilerParams(
        needs_layout_passes=False, use_tc_tiling_on_sc=True),
)
def _sc_main(uid_hbm, iid_hbm, ut_hbm, it_hbm, ut_tail, it_tail, w_hbm,
             part_hbm, ids, slab0, slab1, pcol, pb, bases, res, bidx, wv,
             zerov, shared, sem0, sem1):
    c = lax.axis_index("c")
    s = lax.axis_index("s")
    iota = lax.iota(jnp.int32, D)
    slabs = [slab0, slab1]
    sems = [sem0, sem1]

    def slab_dma(j, issue):
        buf = slabs[j % 2]
        sem = sems[j % 2]
        sidx = j * 16 + s

        def run(src, dst):
            if issue:
                pltpu.async_copy(src, dst, sem)
            else:
                pltpu.make_async_copy(src, dst, sem).wait()

        @pl.when(jnp.logical_and(c == 0, sidx < NFULL))
        def _():
            run(ut_hbm.at[:, pl.ds(sidx * SLAB, SLAB)], buf)

        @pl.when(jnp.logical_and(c == 1, sidx < NFULL))
        def _():
            run(it_hbm.at[:, pl.ds(sidx * SLAB, SLAB)], buf)

        @pl.when(jnp.logical_and(c == 0, sidx == NFULL))
        def _():
            run(ut_tail, buf.at[:, pl.ds(0, TAIL_W)])

        @pl.when(jnp.logical_and(c == 1, sidx == NFULL))
        def _():
            run(it_tail, buf.at[:, pl.ds(0, TAIL_W)])

    slab_dma(0, True)
    slab_dma(1, True)

    @pl.when(c == 0)
    def _():
        pltpu.sync_copy(uid_hbm, ids)

    @pl.when(c == 1)
    def _():
        pltpu.sync_copy(iid_hbm, ids)

    pltpu.sync_copy(w_hbm, wv)

    zeros16 = jnp.zeros((D,), jnp.float32)
    init16 = jnp.where(c == 0, wv[2 * D], zeros16)
    for k in range(1024 // D):
        zerov[pl.ds(k * D, D)] = init16
    pltpu.sync_copy(zerov, shared.at[pl.ds(s * 1024, 1024)])
    for k in range(RCH):
        for t in range(128 // D):
            res[k, pl.ds(t * D, D)] = zeros16
            bidx[k, pl.ds(t * D, D)] = (k * 128 + t * D + iota) % B

    bases[pl.ds(0, D)] = iota * CAP
    bases[pl.ds(D, D)] = (D + iota) * CAP

    def bucket(g, carry):
        idv = ids[pl.ds(g * D, D)]
        gslab = idv >> SHIFT
        mine = (gslab & 15) == s
        bkt = gslab >> 4
        rank, last = plsc.scan_count(bkt, mask=mine)
        basev = plsc.load_gather(bases, [bkt])
        pos = basev + rank - 1
        colrel = idv & (SLAB - 1)
        bvec = g * D + iota
        plsc.store_scatter(pcol, [pos], colrel, mask=mine)
        plsc.store_scatter(pb, [pos], bvec, mask=mine)
        plsc.addupdate_scatter(bases, [bkt], rank, mask=last)
        return carry

    lax.fori_loop(0, B // D, bucket, 0)
    baselo = bases[pl.ds(0, D)]
    basehi = bases[pl.ds(D, D)]

    plsc.subcore_barrier()

    nw = jnp.int32(0)
    for j in range(NBKT):
        buf = slabs[j % 2]
        if j < 16:
            cnt = baselo[j] - j * CAP
        else:
            cnt = basehi[j - 16] - j * CAP
        slab_dma(j, False)

        ng = (cnt + D - 1) >> 4

        def pair_group(g, carry, cnt=cnt, nw=nw, j=j, buf=buf):
            valid = (g * D + iota) < cnt
            colv = pcol[pl.ds(j * CAP + g * D, D)]
            bv = pb[pl.ds(j * CAP + g * D, D)]
            acc = jnp.zeros((D,), jnp.float32)
            for d in range(D):
                dfull = jnp.full((D,), d, jnp.int32)
                wrow = wv[c * D + d]
                acc = acc + plsc.load_gather(
                    buf, [dfull, colv], mask=valid) * wrow
            posv = nw + g * D + iota
            rowv = posv >> 7
            lanev = posv & 127
            plsc.store_scatter(res, [rowv, lanev], acc, mask=valid)
            plsc.store_scatter(bidx, [rowv, lanev], bv, mask=valid)
            return carry

        lax.fori_loop(0, ng, pair_group, 0)
        nw = nw + ng * D
        if j + 2 < NBKT:
            slab_dma(j + 2, True)

    for k in range(RCH):
        @pl.when(k * 128 < nw)
        def _():
            pltpu.async_copy(res.at[k], shared.at[bidx.at[k]], sem0,
                             add=True)
    for k in range(RCH):
        @pl.when(k * 128 < nw)
        def _():
            pltpu.make_async_copy(res.at[k], shared.at[bidx.at[k]],
                                  sem0).wait()

    plsc.subcore_barrier()

    pltpu.sync_copy(shared.at[pl.ds(s * 1024, 1024)],
                    part_hbm.at[pl.ds(c * B + s * 1024, 1024)])


def _tc_combine_body(part_ref, o_ref):
    o_ref[...] = part_ref[pl.ds(0, B)] + part_ref[pl.ds(B, B)]


_tc_combine = pl.pallas_call(
    _tc_combine_body,
    out_shape=jax.ShapeDtypeStruct((B,), jnp.float32),
)


def kernel(user_ids, item_ids, user_table, item_table, fc_w, fc_b):
    uid = user_ids.astype(jnp.int32)
    iid = item_ids.astype(jnp.int32)
    w = jnp.concatenate([fc_w.reshape(2 * D), fc_b.reshape(1)])
    wmat = jnp.broadcast_to(w[:, None], (2 * D + 1, D)).astype(jnp.float32)
    ut_t = user_table.T
    it_t = item_table.T
    ut_tail = jnp.pad(ut_t[:, TAIL_C0:], ((0, 0), (0, TAIL_W - (V - TAIL_C0))))
    it_tail = jnp.pad(it_t[:, TAIL_C0:], ((0, 0), (0, TAIL_W - (V - TAIL_C0))))
    part = _sc_main(uid, iid, ut_t, it_t, ut_tail, it_tail, wmat)
    return _tc_combine(part)
